# baseline (speedup 1.0000x reference)
import functools

import jax
import jax.numpy as jnp
from jax import lax
from jax.experimental import pallas as pl
from jax.experimental.pallas import tpu as pltpu
from jax.experimental.pallas import tpu_sc as plsc

N = 4096
D = 2048
E = 8
K = 2
EP = 128
M = N * K
TILE = 256
NB = M // TILE + E
MPAD = NB * TILE

NW = 32
GROWS = MPAD // NW
GCH = 40
GNCH = GROWS // GCH
CTOK = N // NW
CCH = 8
CNCH = CTOK // CCH


def _gating_body(x_ref, wg_ref, bg_ref, idx_ref, gate_ref, xb_ref):
    x = x_ref[...]
    xb_ref[...] = x.astype(jnp.bfloat16)
    wg = wg_ref[...]
    logits = lax.dot_general(
        x, wg, (((1,), (1,)), ((), ())),
        preferred_element_type=jnp.float32,
        precision=lax.Precision.DEFAULT,
    ) + bg_ref[...]
    lane = lax.broadcasted_iota(jnp.int32, logits.shape, 1)
    v0 = jnp.max(logits, axis=1, keepdims=True)
    i0 = jnp.min(jnp.where(logits == v0, lane, EP), axis=1, keepdims=True)
    l2 = jnp.where(lane == i0, jnp.float32(-1e30), logits)
    v1 = jnp.max(l2, axis=1, keepdims=True)
    i1 = jnp.min(jnp.where(l2 == v1, lane, EP), axis=1, keepdims=True)
    t = jnp.exp(v1 - v0)
    g0 = 1.0 / (1.0 + t)
    g1 = t / (1.0 + t)
    idx_ref[...] = jnp.where(lane == 0, i0, jnp.where(lane == 1, i1, 0))
    gate_ref[...] = jnp.where(lane == 0, g0, jnp.where(lane == 1, g1, 0.0))


def _gating(inp, Wg, bg):
    wgp = jnp.zeros((EP, D), jnp.float32).at[:E].set(Wg)
    bgp = jnp.full((1, EP), -1e30, jnp.float32).at[0, :E].set(bg)
    bn = 1024
    idx_out, gate_out, inp_bf16 = pl.pallas_call(
        _gating_body,
        grid=(N // bn,),
        in_specs=[
            pl.BlockSpec((bn, D), lambda b: (b, 0)),
            pl.BlockSpec((EP, D), lambda b: (0, 0)),
            pl.BlockSpec((1, EP), lambda b: (0, 0)),
        ],
        out_specs=[
            pl.BlockSpec((bn, EP), lambda b: (b, 0)),
            pl.BlockSpec((bn, EP), lambda b: (b, 0)),
            pl.BlockSpec((bn, D), lambda b: (b, 0)),
        ],
        out_shape=[
            jax.ShapeDtypeStruct((N, EP), jnp.int32),
            jax.ShapeDtypeStruct((N, EP), jnp.float32),
            jax.ShapeDtypeStruct((N, D), jnp.bfloat16),
        ],
    )(inp, wgp, bgp)
    return idx_out[:, :K], gate_out[:, :K], inp_bf16


def _gather_body(src_hbm, gidx_hbm, out_hbm, idx0, idx1, rows_v, sem0, sem1):
    wid = lax.axis_index("s") * 2 + lax.axis_index("c")
    base = wid * GROWS
    idxs = (idx0, idx1)
    sems = (sem0, sem1)
    pending = [None, None]
    pltpu.sync_copy(gidx_hbm.at[pl.ds(base, GCH)], idx0)
    pending[0] = pltpu.async_copy(src_hbm.at[idx0], rows_v.at[0], sem0)
    for c in range(GNCH):
        b = c % 2
        nb = (c + 1) % 2
        if c + 1 < GNCH:
            pltpu.sync_copy(
                gidx_hbm.at[pl.ds(base + (c + 1) * GCH, GCH)], idxs[nb])
            pending[nb] = pltpu.async_copy(
                src_hbm.at[idxs[nb]], rows_v.at[nb], sems[nb])
        pending[b].wait()
        pltpu.sync_copy(rows_v.at[b], out_hbm.at[pl.ds(base + c * GCH, GCH)])


def _dispatch_gather(inp_bf16, gather_src):
    packed = jax.lax.bitcast_convert_type(
        inp_bf16.reshape(N, D // 2, 2), jnp.int32)
    mesh = plsc.VectorSubcoreMesh(core_axis_name="c", subcore_axis_name="s")
    fn = pl.kernel(
        _gather_body,
        out_type=jax.ShapeDtypeStruct((MPAD, D // 2), jnp.int32),
        mesh=mesh,
        scratch_types=[
            pltpu.VMEM((GCH,), jnp.int32),
            pltpu.VMEM((GCH,), jnp.int32),
            pltpu.VMEM((2, GCH, D // 2), jnp.int32),
            pltpu.SemaphoreType.DMA,
            pltpu.SemaphoreType.DMA,
        ],
    )
    out = fn(packed, gather_src)
    return jax.lax.bitcast_convert_type(out, jnp.bfloat16).reshape(MPAD, D)


def _gemm_body(bw_ref, act_ref, x_ref, w_ref, b_ref, g_ref, y_ref):
    del bw_ref
    blk = pl.program_id(0)

    @pl.when(act_ref[blk] > 0)
    def _():
        x = x_ref[...].astype(jnp.float32)
        w = w_ref[0]
        acc = lax.dot_general(
            x, w, (((1,), (1,)), ((), ())),
            preferred_element_type=jnp.float32,
            precision=lax.Precision.DEFAULT,
        )
        y_ref[...] = (acc + b_ref[0]) * g_ref[...]


def _grouped_gemm(Xg, We, be, blk_weight, blk_active, slot_gate):
    grid_spec = pltpu.PrefetchScalarGridSpec(
        num_scalar_prefetch=2,
        grid=(NB,),
        in_specs=[
            pl.BlockSpec((TILE, D), lambda b, s, a: (b, 0)),
            pl.BlockSpec((1, D, D), lambda b, s, a: (s[b], 0, 0)),
            pl.BlockSpec((1, 1, D), lambda b, s, a: (s[b], 0, 0)),
            pl.BlockSpec((TILE, 1), lambda b, s, a: (b, 0)),
        ],
        out_specs=pl.BlockSpec((TILE, D), lambda b, s, a: (b, 0)),
    )
    return pl.pallas_call(
        _gemm_body,
        grid_spec=grid_spec,
        out_shape=jax.ShapeDtypeStruct((MPAD, D), jnp.float32),
    )(blk_weight, blk_active, Xg, We, be.reshape(E, 1, D),
      slot_gate.reshape(MPAD, 1))


def _combine_body(y_hbm, pos_hbm, out_hbm, idx0, idx1, rows_v, out_v,
                  sem0, sem1):
    wid = lax.axis_index("s") * 2 + lax.axis_index("c")
    base = wid * CTOK
    idxs = (idx0, idx1)
    sems = (sem0, sem1)
    pending = [None, None]
    pltpu.sync_copy(pos_hbm.at[pl.ds(K * base, K * CCH)], idx0)
    pending[0] = pltpu.async_copy(y_hbm.at[idx0], rows_v.at[0], sem0)
    for c in range(CNCH):
        b = c % 2
        nb = (c + 1) % 2
        if c + 1 < CNCH:
            pltpu.sync_copy(
                pos_hbm.at[pl.ds(K * (base + (c + 1) * CCH), K * CCH)],
                idxs[nb])
            pending[nb] = pltpu.async_copy(
                y_hbm.at[idxs[nb]], rows_v.at[nb], sems[nb])
        pending[b].wait()

        def jbody(j, carry):
            off = j * 16
            for t in range(CCH):
                a = rows_v[b, 2 * t, pl.ds(off, 16)]
                bb = rows_v[b, 2 * t + 1, pl.ds(off, 16)]
                out_v[b, t, pl.ds(off, 16)] = a + bb
            return carry

        lax.fori_loop(0, D // 16, jbody, 0)
        pltpu.sync_copy(out_v.at[b], out_hbm.at[pl.ds(base + c * CCH, CCH)])


def _combine(Y, pos):
    mesh = plsc.VectorSubcoreMesh(core_axis_name="c", subcore_axis_name="s")
    fn = pl.kernel(
        _combine_body,
        out_type=jax.ShapeDtypeStruct((N, D), jnp.float32),
        mesh=mesh,
        scratch_types=[
            pltpu.VMEM((K * CCH,), jnp.int32),
            pltpu.VMEM((K * CCH,), jnp.int32),
            pltpu.VMEM((2, K * CCH, D), jnp.float32),
            pltpu.VMEM((2, CCH, D), jnp.float32),
            pltpu.SemaphoreType.DMA,
            pltpu.SemaphoreType.DMA,
        ],
    )
    return fn(Y, pos)


def _route(idx2, gates2):
    e_flat = idx2.reshape(M)
    g_flat = gates2.reshape(M)
    onehot = (e_flat[:, None] == jnp.arange(E, dtype=jnp.int32)[None, :])
    oh = onehot.astype(jnp.int32)
    counts = jnp.sum(oh, axis=0)
    rank = jnp.sum(jnp.where(onehot, jnp.cumsum(oh, axis=0) - oh, 0), axis=1)
    nblk = (counts + TILE - 1) // TILE
    cum = jnp.cumsum(nblk)
    blk_off = cum - nblk
    pos = blk_off[e_flat] * TILE + rank
    gather_src = jnp.zeros((MPAD,), jnp.int32).at[pos].set(
        jnp.arange(M, dtype=jnp.int32) // K)
    slot_gate = jnp.zeros((MPAD,), jnp.float32).at[pos].set(g_flat)
    bids = jnp.arange(NB, dtype=jnp.int32)
    blk_exp = jnp.minimum(
        jnp.sum((bids[:, None] >= cum[None, :]).astype(jnp.int32), axis=1),
        E - 1)
    blk_active = (bids < cum[E - 1]).astype(jnp.int32)
    last_exp = jnp.max(jnp.where(counts > 0,
                                 jnp.arange(E, dtype=jnp.int32), 0))
    blk_weight = jnp.where(blk_active > 0, blk_exp, last_exp)
    return gather_src, slot_gate, blk_weight, blk_active, pos


def kernel(x, y, We, be, Wg, bg):
    inp = jnp.concatenate([x, y], axis=1)
    idx2, gates2, inp_bf16 = _gating(inp, Wg, bg)
    gather_src, slot_gate, blk_weight, blk_active, pos = _route(idx2, gates2)
    Xg = _dispatch_gather(inp_bf16, gather_src)
    Y = _grouped_gemm(Xg, We, be, blk_weight, blk_active, slot_gate)
    return _combine(Y, pos)


# speedup vs baseline: 2.0030x; 2.0030x over previous
import functools

import jax
import jax.numpy as jnp
from jax import lax
from jax.experimental import pallas as pl
from jax.experimental.pallas import tpu as pltpu
from jax.experimental.pallas import tpu_sc as plsc

N = 4096
D = 2048
E = 8
K = 2
EP = 128
M = N * K
TILE = 256
NB = M // TILE + E
MPAD = NB * TILE

NW = 32
GROWS = MPAD // NW
GCH = 40
GNCH = GROWS // GCH
CTOK = N // NW
CCH = 8
CNCH = CTOK // CCH


def _gating_body(x_ref, wg_ref, bg_ref, idx_ref, gate_ref, xb_ref):
    x = x_ref[...]
    lo = jax.lax.bitcast_convert_type(
        x[:, :D // 2].astype(jnp.bfloat16).astype(jnp.float32), jnp.uint32)
    hi = jax.lax.bitcast_convert_type(
        x[:, D // 2:].astype(jnp.bfloat16).astype(jnp.float32), jnp.uint32)
    xb_ref[...] = ((lo >> 16) | hi).astype(jnp.int32)
    wg = wg_ref[...]
    logits = lax.dot_general(
        x, wg, (((1,), (1,)), ((), ())),
        preferred_element_type=jnp.float32,
        precision=lax.Precision.DEFAULT,
    ) + bg_ref[...]
    lane = lax.broadcasted_iota(jnp.int32, logits.shape, 1)
    v0 = jnp.max(logits, axis=1, keepdims=True)
    i0 = jnp.min(jnp.where(logits == v0, lane, EP), axis=1, keepdims=True)
    l2 = jnp.where(lane == i0, jnp.float32(-1e30), logits)
    v1 = jnp.max(l2, axis=1, keepdims=True)
    i1 = jnp.min(jnp.where(l2 == v1, lane, EP), axis=1, keepdims=True)
    t = jnp.exp(v1 - v0)
    g0 = 1.0 / (1.0 + t)
    g1 = t / (1.0 + t)
    idx_ref[...] = jnp.where(lane == 0, i0, jnp.where(lane == 1, i1, 0))
    gate_ref[...] = jnp.where(lane == 0, g0, jnp.where(lane == 1, g1, 0.0))


def _gating(inp, Wg, bg):
    wgp = jnp.zeros((EP, D), jnp.float32).at[:E].set(Wg)
    bgp = jnp.full((1, EP), -1e30, jnp.float32).at[0, :E].set(bg)
    bn = 1024
    idx_out, gate_out, inp_bf16 = pl.pallas_call(
        _gating_body,
        grid=(N // bn,),
        in_specs=[
            pl.BlockSpec((bn, D), lambda b: (b, 0)),
            pl.BlockSpec((EP, D), lambda b: (0, 0)),
            pl.BlockSpec((1, EP), lambda b: (0, 0)),
        ],
        out_specs=[
            pl.BlockSpec((bn, EP), lambda b: (b, 0)),
            pl.BlockSpec((bn, EP), lambda b: (b, 0)),
            pl.BlockSpec((bn, D // 2), lambda b: (b, 0)),
        ],
        out_shape=[
            jax.ShapeDtypeStruct((N, EP), jnp.int32),
            jax.ShapeDtypeStruct((N, EP), jnp.float32),
            jax.ShapeDtypeStruct((N, D // 2), jnp.int32),
        ],
    )(inp, wgp, bgp)
    return idx_out[:, :K], gate_out[:, :K], inp_bf16


def _gather_body(src_hbm, gidx_hbm, out_hbm, idx0, idx1, rows_v, sem0, sem1):
    wid = lax.axis_index("s") * 2 + lax.axis_index("c")
    base = wid * GROWS
    idxs = (idx0, idx1)
    sems = (sem0, sem1)
    pending = [None, None]
    pltpu.sync_copy(gidx_hbm.at[pl.ds(base, GCH)], idx0)
    pending[0] = pltpu.async_copy(src_hbm.at[idx0], rows_v.at[0], sem0)
    for c in range(GNCH):
        b = c % 2
        nb = (c + 1) % 2
        if c + 1 < GNCH:
            pltpu.sync_copy(
                gidx_hbm.at[pl.ds(base + (c + 1) * GCH, GCH)], idxs[nb])
            pending[nb] = pltpu.async_copy(
                src_hbm.at[idxs[nb]], rows_v.at[nb], sems[nb])
        pending[b].wait()
        pltpu.sync_copy(rows_v.at[b], out_hbm.at[pl.ds(base + c * GCH, GCH)])


def _dispatch_gather(inp_packed, gather_src):
    mesh = plsc.VectorSubcoreMesh(core_axis_name="c", subcore_axis_name="s")
    fn = pl.kernel(
        _gather_body,
        out_type=jax.ShapeDtypeStruct((MPAD, D // 2), jnp.int32),
        mesh=mesh,
        scratch_types=[
            pltpu.VMEM((GCH,), jnp.int32),
            pltpu.VMEM((GCH,), jnp.int32),
            pltpu.VMEM((2, GCH, D // 2), jnp.int32),
            pltpu.SemaphoreType.DMA,
            pltpu.SemaphoreType.DMA,
        ],
    )
    return fn(inp_packed, gather_src)


def _gemm_body(bw_ref, act_ref, x_ref, w_ref, b_ref, g_ref, y_ref):
    del bw_ref
    blk = pl.program_id(0)

    @pl.when(act_ref[blk] > 0)
    def _():
        u = jax.lax.bitcast_convert_type(x_ref[...], jnp.uint32)
        x_lo = jax.lax.bitcast_convert_type(u << 16, jnp.float32)
        x_hi = jax.lax.bitcast_convert_type(
            u & jnp.uint32(0xFFFF0000), jnp.float32)
        w = w_ref[0]
        dn = (((1,), (1,)), ((), ()))
        acc = lax.dot_general(
            x_lo, w[:, :D // 2], dn,
            preferred_element_type=jnp.float32,
            precision=lax.Precision.DEFAULT,
        ) + lax.dot_general(
            x_hi, w[:, D // 2:], dn,
            preferred_element_type=jnp.float32,
            precision=lax.Precision.DEFAULT,
        )
        y_ref[...] = (acc + b_ref[0]) * g_ref[...]


def _grouped_gemm(Xg, We, be, blk_weight, blk_active, slot_gate):
    grid_spec = pltpu.PrefetchScalarGridSpec(
        num_scalar_prefetch=2,
        grid=(NB,),
        in_specs=[
            pl.BlockSpec((TILE, D // 2), lambda b, s, a: (b, 0)),
            pl.BlockSpec((1, D, D), lambda b, s, a: (s[b], 0, 0)),
            pl.BlockSpec((1, 1, D), lambda b, s, a: (s[b], 0, 0)),
            pl.BlockSpec((TILE, 1), lambda b, s, a: (b, 0)),
        ],
        out_specs=pl.BlockSpec((TILE, D), lambda b, s, a: (b, 0)),
    )
    return pl.pallas_call(
        _gemm_body,
        grid_spec=grid_spec,
        out_shape=jax.ShapeDtypeStruct((MPAD, D), jnp.float32),
    )(blk_weight, blk_active, Xg, We, be.reshape(E, 1, D),
      slot_gate.reshape(MPAD, 1))


def _combine_body(y_hbm, pos_hbm, out_hbm, idx0, idx1, rows_v, out_v,
                  sem0, sem1):
    wid = lax.axis_index("s") * 2 + lax.axis_index("c")
    base = wid * CTOK
    idxs = (idx0, idx1)
    sems = (sem0, sem1)
    pending = [None, None]
    pltpu.sync_copy(pos_hbm.at[pl.ds(K * base, K * CCH)], idx0)
    pending[0] = pltpu.async_copy(y_hbm.at[idx0], rows_v.at[0], sem0)
    for c in range(CNCH):
        b = c % 2
        nb = (c + 1) % 2
        if c + 1 < CNCH:
            pltpu.sync_copy(
                pos_hbm.at[pl.ds(K * (base + (c + 1) * CCH), K * CCH)],
                idxs[nb])
            pending[nb] = pltpu.async_copy(
                y_hbm.at[idxs[nb]], rows_v.at[nb], sems[nb])
        pending[b].wait()

        def jbody(j, carry):
            off = j * 16
            for t in range(CCH):
                a = rows_v[b, 2 * t, pl.ds(off, 16)]
                bb = rows_v[b, 2 * t + 1, pl.ds(off, 16)]
                out_v[b, t, pl.ds(off, 16)] = a + bb
            return carry

        lax.fori_loop(0, D // 16, jbody, 0)
        pltpu.sync_copy(out_v.at[b], out_hbm.at[pl.ds(base + c * CCH, CCH)])


def _combine(Y, pos):
    mesh = plsc.VectorSubcoreMesh(core_axis_name="c", subcore_axis_name="s")
    fn = pl.kernel(
        _combine_body,
        out_type=jax.ShapeDtypeStruct((N, D), jnp.float32),
        mesh=mesh,
        scratch_types=[
            pltpu.VMEM((K * CCH,), jnp.int32),
            pltpu.VMEM((K * CCH,), jnp.int32),
            pltpu.VMEM((2, K * CCH, D), jnp.float32),
            pltpu.VMEM((2, CCH, D), jnp.float32),
            pltpu.SemaphoreType.DMA,
            pltpu.SemaphoreType.DMA,
        ],
    )
    return fn(Y, pos)


def _route(idx2, gates2):
    e_flat = idx2.reshape(M)
    g_flat = gates2.reshape(M)
    onehot = (e_flat[:, None] == jnp.arange(E, dtype=jnp.int32)[None, :])
    oh = onehot.astype(jnp.int32)
    counts = jnp.sum(oh, axis=0)
    rank = jnp.sum(jnp.where(onehot, jnp.cumsum(oh, axis=0) - oh, 0), axis=1)
    nblk = (counts + TILE - 1) // TILE
    cum = jnp.cumsum(nblk)
    blk_off = cum - nblk
    pos = blk_off[e_flat] * TILE + rank
    gather_src = jnp.zeros((MPAD,), jnp.int32).at[pos].set(
        jnp.arange(M, dtype=jnp.int32) // K)
    slot_gate = jnp.zeros((MPAD,), jnp.float32).at[pos].set(g_flat)
    bids = jnp.arange(NB, dtype=jnp.int32)
    blk_exp = jnp.minimum(
        jnp.sum((bids[:, None] >= cum[None, :]).astype(jnp.int32), axis=1),
        E - 1)
    blk_active = (bids < cum[E - 1]).astype(jnp.int32)
    last_exp = jnp.max(jnp.where(counts > 0,
                                 jnp.arange(E, dtype=jnp.int32), 0))
    blk_weight = jnp.where(blk_active > 0, blk_exp, last_exp)
    return gather_src, slot_gate, blk_weight, blk_active, pos


def kernel(x, y, We, be, Wg, bg):
    inp = jnp.concatenate([x, y], axis=1)
    idx2, gates2, inp_bf16 = _gating(inp, Wg, bg)
    gather_src, slot_gate, blk_weight, blk_active, pos = _route(idx2, gates2)
    Xg = _dispatch_gather(inp_bf16, gather_src)
    Y = _grouped_gemm(Xg, We, be, blk_weight, blk_active, slot_gate)
    return _combine(Y, pos)


# speedup vs baseline: 3.5797x; 1.7872x over previous
import functools

import jax
import jax.numpy as jnp
from jax import lax
from jax.experimental import pallas as pl
from jax.experimental.pallas import tpu as pltpu
from jax.experimental.pallas import tpu_sc as plsc

N = 4096
D = 2048
H = D // 2
E = 8
K = 2
EP = 128
M = N * K
TILE = 256
NB = M // TILE + E
MPAD = NB * TILE

NW = 32
TWORK = N // NW
TCH = 16
TNCH = TWORK // TCH
CCH = 8
CNCH = TWORK // CCH


def _gating_body(x_ref, y_ref, wg_ref, bg_ref, idx_ref, gate_ref, xb_ref):
    x = x_ref[...]
    y = y_ref[...]
    wg = wg_ref[...]
    dn = (((1,), (1,)), ((), ()))
    logits = lax.dot_general(
        x, wg[:, :H], dn,
        preferred_element_type=jnp.float32,
        precision=lax.Precision.DEFAULT,
    ) + lax.dot_general(
        y, wg[:, H:], dn,
        preferred_element_type=jnp.float32,
        precision=lax.Precision.DEFAULT,
    ) + bg_ref[...]
    lane = lax.broadcasted_iota(jnp.int32, logits.shape, 1)
    v0 = jnp.max(logits, axis=1, keepdims=True)
    i0 = jnp.min(jnp.where(logits == v0, lane, EP), axis=1, keepdims=True)
    l2 = jnp.where(lane == i0, jnp.float32(-1e30), logits)
    v1 = jnp.max(l2, axis=1, keepdims=True)
    i1 = jnp.min(jnp.where(l2 == v1, lane, EP), axis=1, keepdims=True)
    t = jnp.exp(v1 - v0)
    g0 = 1.0 / (1.0 + t)
    g1 = t / (1.0 + t)
    idx_ref[...] = jnp.where(lane == 0, i0, jnp.where(lane == 1, i1, 0))
    gate_ref[...] = jnp.where(lane == 0, g0, jnp.where(lane == 1, g1, 0.0))
    lo = jax.lax.bitcast_convert_type(
        x.astype(jnp.bfloat16).astype(jnp.float32), jnp.uint32)
    hi = jax.lax.bitcast_convert_type(
        y.astype(jnp.bfloat16).astype(jnp.float32), jnp.uint32)
    xb_ref[...] = ((lo >> 16) | hi).astype(jnp.int32)


def _gating(x, y, Wg, bg):
    wgp = jnp.zeros((EP, D), jnp.float32).at[:E].set(Wg)
    bgp = jnp.full((1, EP), -1e30, jnp.float32).at[0, :E].set(bg)
    bn = 1024
    idx_out, gate_out, packed = pl.pallas_call(
        _gating_body,
        grid=(N // bn,),
        in_specs=[
            pl.BlockSpec((bn, H), lambda b: (b, 0)),
            pl.BlockSpec((bn, H), lambda b: (b, 0)),
            pl.BlockSpec((EP, D), lambda b: (0, 0)),
            pl.BlockSpec((1, EP), lambda b: (0, 0)),
        ],
        out_specs=[
            pl.BlockSpec((bn, EP), lambda b: (b, 0)),
            pl.BlockSpec((bn, EP), lambda b: (b, 0)),
            pl.BlockSpec((bn, H), lambda b: (b, 0)),
        ],
        out_shape=[
            jax.ShapeDtypeStruct((N, EP), jnp.int32),
            jax.ShapeDtypeStruct((N, EP), jnp.float32),
            jax.ShapeDtypeStruct((N, H), jnp.int32),
        ],
    )(x, y, wgp, bgp)
    return idx_out[:, :K], gate_out[:, :K], packed


def _dispatch_body(src_hbm, p0_hbm, p1_hbm, out_hbm,
                   i0a, i0b, i1a, i1b, rows_v,
                   s0a, s0b, s1a, s1b):
    wid = lax.axis_index("s") * 2 + lax.axis_index("c")
    tbase = wid * TWORK
    idx0 = (i0a, i0b)
    idx1 = (i1a, i1b)
    sem0 = (s0a, s0b)
    sem1 = (s1a, s1b)
    pend = [None, None, None, None]
    for c in range(TNCH):
        b = c % 2
        if c >= 2:
            pend[2 * b].wait()
            pend[2 * b + 1].wait()
        t0 = tbase + c * TCH
        pltpu.sync_copy(src_hbm.at[pl.ds(t0, TCH)], rows_v.at[b])
        pltpu.sync_copy(p0_hbm.at[pl.ds(t0, TCH)], idx0[b])
        pltpu.sync_copy(p1_hbm.at[pl.ds(t0, TCH)], idx1[b])
        pend[2 * b] = pltpu.async_copy(
            rows_v.at[b], out_hbm.at[idx0[b]], sem0[b])
        pend[2 * b + 1] = pltpu.async_copy(
            rows_v.at[b], out_hbm.at[idx1[b]], sem1[b])
    for h in pend:
        h.wait()


def _dispatch(packed, pos0, pos1):
    mesh = plsc.VectorSubcoreMesh(core_axis_name="c", subcore_axis_name="s")
    fn = pl.kernel(
        _dispatch_body,
        out_type=jax.ShapeDtypeStruct((MPAD, H), jnp.int32),
        mesh=mesh,
        scratch_types=[
            pltpu.VMEM((TCH,), jnp.int32),
            pltpu.VMEM((TCH,), jnp.int32),
            pltpu.VMEM((TCH,), jnp.int32),
            pltpu.VMEM((TCH,), jnp.int32),
            pltpu.VMEM((2, TCH, H), jnp.int32),
            pltpu.SemaphoreType.DMA,
            pltpu.SemaphoreType.DMA,
            pltpu.SemaphoreType.DMA,
            pltpu.SemaphoreType.DMA,
        ],
    )
    return fn(packed, pos0, pos1)


def _gemm_body(bw_ref, act_ref, x_ref, w_ref, b_ref, y_ref):
    del bw_ref
    blk = pl.program_id(0)

    @pl.when(act_ref[blk] > 0)
    def _():
        u = jax.lax.bitcast_convert_type(x_ref[...], jnp.uint32)
        x_lo = jax.lax.bitcast_convert_type(u << 16, jnp.float32)
        x_hi = jax.lax.bitcast_convert_type(
            u & jnp.uint32(0xFFFF0000), jnp.float32)
        w = w_ref[0]
        dn = (((1,), (1,)), ((), ()))
        acc = lax.dot_general(
            x_lo, w[:, :H], dn,
            preferred_element_type=jnp.float32,
            precision=lax.Precision.DEFAULT,
        ) + lax.dot_general(
            x_hi, w[:, H:], dn,
            preferred_element_type=jnp.float32,
            precision=lax.Precision.DEFAULT,
        )
        y_ref[...] = acc + b_ref[0]


def _grouped_gemm(Xg, We, be, blk_weight, blk_active):
    grid_spec = pltpu.PrefetchScalarGridSpec(
        num_scalar_prefetch=2,
        grid=(NB,),
        in_specs=[
            pl.BlockSpec((TILE, H), lambda b, s, a: (b, 0)),
            pl.BlockSpec((1, D, D), lambda b, s, a: (s[b], 0, 0)),
            pl.BlockSpec((1, 1, D), lambda b, s, a: (s[b], 0, 0)),
        ],
        out_specs=pl.BlockSpec((TILE, D), lambda b, s, a: (b, 0)),
    )
    return pl.pallas_call(
        _gemm_body,
        grid_spec=grid_spec,
        out_shape=jax.ShapeDtypeStruct((MPAD, D), jnp.float32),
    )(blk_weight, blk_active, Xg, We, be.reshape(E, 1, D))


def _combine_body(y_hbm, pos_hbm, g_hbm, out_hbm,
                  idx0, idx1, gv, rows_v, out_v, sem0, sem1):
    wid = lax.axis_index("s") * 2 + lax.axis_index("c")
    base = wid * TWORK
    idxs = (idx0, idx1)
    sems = (sem0, sem1)
    pltpu.sync_copy(g_hbm.at[pl.ds(K * base, K * TWORK)], gv)
    pend = [None, None]
    pltpu.sync_copy(pos_hbm.at[pl.ds(K * base, K * CCH)], idx0)
    pend[0] = pltpu.async_copy(y_hbm.at[idx0], rows_v.at[0], sem0)
    for c in range(CNCH):
        b = c % 2
        nb = (c + 1) % 2
        if c + 1 < CNCH:
            pltpu.sync_copy(
                pos_hbm.at[pl.ds(K * (base + (c + 1) * CCH), K * CCH)],
                idxs[nb])
            pend[nb] = pltpu.async_copy(
                y_hbm.at[idxs[nb]], rows_v.at[nb], sems[nb])
        pend[b].wait()
        gvec = gv[pl.ds(K * c * CCH, K * CCH)]

        def jbody(j, carry):
            off = j * 16
            for t in range(CCH):
                g0 = gvec[2 * t]
                g1 = gvec[2 * t + 1]
                a = rows_v[b, 2 * t, pl.ds(off, 16)]
                bb = rows_v[b, 2 * t + 1, pl.ds(off, 16)]
                out_v[b, t, pl.ds(off, 16)] = a * g0 + bb * g1
            return carry

        lax.fori_loop(0, D // 16, jbody, 0)
        pltpu.sync_copy(out_v.at[b], out_hbm.at[pl.ds(base + c * CCH, CCH)])


def _combine(Y, pos, gates):
    mesh = plsc.VectorSubcoreMesh(core_axis_name="c", subcore_axis_name="s")
    fn = pl.kernel(
        _combine_body,
        out_type=jax.ShapeDtypeStruct((N, D), jnp.float32),
        mesh=mesh,
        scratch_types=[
            pltpu.VMEM((K * CCH,), jnp.int32),
            pltpu.VMEM((K * CCH,), jnp.int32),
            pltpu.VMEM((K * TWORK,), jnp.float32),
            pltpu.VMEM((2, K * CCH, D), jnp.float32),
            pltpu.VMEM((2, CCH, D), jnp.float32),
            pltpu.SemaphoreType.DMA,
            pltpu.SemaphoreType.DMA,
        ],
    )
    return fn(Y, pos, gates)


def _route(idx2):
    e_flat = idx2.reshape(M)
    onehot = (e_flat[:, None] == jnp.arange(E, dtype=jnp.int32)[None, :])
    oh = onehot.astype(jnp.int32)
    counts = jnp.sum(oh, axis=0)
    rank = jnp.sum(jnp.where(onehot, jnp.cumsum(oh, axis=0) - oh, 0), axis=1)
    nblk = (counts + TILE - 1) // TILE
    cum = jnp.cumsum(nblk)
    blk_off = cum - nblk
    pos = blk_off[e_flat] * TILE + rank
    bids = jnp.arange(NB, dtype=jnp.int32)
    blk_exp = jnp.minimum(
        jnp.sum((bids[:, None] >= cum[None, :]).astype(jnp.int32), axis=1),
        E - 1)
    blk_active = (bids < cum[E - 1]).astype(jnp.int32)
    last_exp = jnp.max(jnp.where(counts > 0,
                                 jnp.arange(E, dtype=jnp.int32), 0))
    blk_weight = jnp.where(blk_active > 0, blk_exp, last_exp)
    pos2 = pos.reshape(N, K)
    return pos, pos2[:, 0], pos2[:, 1], blk_weight, blk_active


def kernel(x, y, We, be, Wg, bg):
    idx2, gates2, packed = _gating(x, y, Wg, bg)
    pos, pos0, pos1, blk_weight, blk_active = _route(idx2)
    Xg = _dispatch(packed, pos0, pos1)
    Y = _grouped_gemm(Xg, We, be, blk_weight, blk_active)
    return _combine(Y, pos, gates2.reshape(M))


# speedup vs baseline: 3.7393x; 1.0446x over previous
import functools

import jax
import jax.numpy as jnp
from jax import lax
from jax.experimental import pallas as pl
from jax.experimental.pallas import tpu as pltpu
from jax.experimental.pallas import tpu_sc as plsc

N = 4096
D = 2048
H = D // 2
E = 8
K = 2
EP = 128
M = N * K
TILE = 256
NB = M // TILE + E
MPAD = NB * TILE

NW = 32
TWORK = N // NW
TCH = 16
TNCH = TWORK // TCH
CCH = 8
CNCH = TWORK // CCH


def _gating_body(x_ref, y_ref, wg_ref, bg_ref, idx_ref, gate_ref, xb_ref):
    x = x_ref[...]
    y = y_ref[...]
    wg = wg_ref[...]
    dn = (((1,), (1,)), ((), ()))
    logits = lax.dot_general(
        x, wg[:, :H], dn,
        preferred_element_type=jnp.float32,
        precision=lax.Precision.DEFAULT,
    ) + lax.dot_general(
        y, wg[:, H:], dn,
        preferred_element_type=jnp.float32,
        precision=lax.Precision.DEFAULT,
    ) + bg_ref[...]
    lane = lax.broadcasted_iota(jnp.int32, logits.shape, 1)
    v0 = jnp.max(logits, axis=1, keepdims=True)
    i0 = jnp.min(jnp.where(logits == v0, lane, EP), axis=1, keepdims=True)
    l2 = jnp.where(lane == i0, jnp.float32(-1e30), logits)
    v1 = jnp.max(l2, axis=1, keepdims=True)
    i1 = jnp.min(jnp.where(l2 == v1, lane, EP), axis=1, keepdims=True)
    t = jnp.exp(v1 - v0)
    g0 = 1.0 / (1.0 + t)
    g1 = t / (1.0 + t)
    idx_ref[...] = jnp.where(lane == 0, i0, jnp.where(lane == 1, i1, 0))
    gate_ref[...] = jnp.where(lane == 0, g0, jnp.where(lane == 1, g1, 0.0))
    lo = jax.lax.bitcast_convert_type(
        x.astype(jnp.bfloat16).astype(jnp.float32), jnp.uint32)
    hi = jax.lax.bitcast_convert_type(
        y.astype(jnp.bfloat16).astype(jnp.float32), jnp.uint32)
    xb_ref[...] = ((lo >> 16) | hi).astype(jnp.int32)


def _gating(x, y, Wg, bg):
    wgp = jnp.zeros((EP, D), jnp.float32).at[:E].set(Wg)
    bgp = jnp.full((1, EP), -1e30, jnp.float32).at[0, :E].set(bg)
    bn = 1024
    idx_out, gate_out, packed = pl.pallas_call(
        _gating_body,
        grid=(N // bn,),
        in_specs=[
            pl.BlockSpec((bn, H), lambda b: (b, 0)),
            pl.BlockSpec((bn, H), lambda b: (b, 0)),
            pl.BlockSpec((EP, D), lambda b: (0, 0)),
            pl.BlockSpec((1, EP), lambda b: (0, 0)),
        ],
        out_specs=[
            pl.BlockSpec((bn, EP), lambda b: (b, 0)),
            pl.BlockSpec((bn, EP), lambda b: (b, 0)),
            pl.BlockSpec((bn, H), lambda b: (b, 0)),
        ],
        out_shape=[
            jax.ShapeDtypeStruct((N, EP), jnp.int32),
            jax.ShapeDtypeStruct((N, EP), jnp.float32),
            jax.ShapeDtypeStruct((N, H), jnp.int32),
        ],
    )(x, y, wgp, bgp)
    return idx_out[:, :K], gate_out[:, :K], packed


def _dispatch_body(src_hbm, p0_hbm, p1_hbm, out_hbm,
                   i0a, i0b, i1a, i1b, rows_v,
                   s0a, s0b, s1a, s1b):
    wid = lax.axis_index("s") * 2 + lax.axis_index("c")
    tbase = wid * TWORK
    idx0 = (i0a, i0b)
    idx1 = (i1a, i1b)
    sem0 = (s0a, s0b)
    sem1 = (s1a, s1b)
    pend = [None, None, None, None]
    for c in range(TNCH):
        b = c % 2
        if c >= 2:
            pend[2 * b].wait()
            pend[2 * b + 1].wait()
        t0 = tbase + c * TCH
        pltpu.sync_copy(src_hbm.at[pl.ds(t0, TCH)], rows_v.at[b])
        pltpu.sync_copy(p0_hbm.at[pl.ds(t0, TCH)], idx0[b])
        pltpu.sync_copy(p1_hbm.at[pl.ds(t0, TCH)], idx1[b])
        pend[2 * b] = pltpu.async_copy(
            rows_v.at[b], out_hbm.at[idx0[b]], sem0[b])
        pend[2 * b + 1] = pltpu.async_copy(
            rows_v.at[b], out_hbm.at[idx1[b]], sem1[b])
    for h in pend:
        h.wait()


def _dispatch(packed, pos0, pos1):
    mesh = plsc.VectorSubcoreMesh(core_axis_name="c", subcore_axis_name="s")
    fn = pl.kernel(
        _dispatch_body,
        out_type=jax.ShapeDtypeStruct((MPAD, H), jnp.int32),
        mesh=mesh,
        scratch_types=[
            pltpu.VMEM((TCH,), jnp.int32),
            pltpu.VMEM((TCH,), jnp.int32),
            pltpu.VMEM((TCH,), jnp.int32),
            pltpu.VMEM((TCH,), jnp.int32),
            pltpu.VMEM((2, TCH, H), jnp.int32),
            pltpu.SemaphoreType.DMA,
            pltpu.SemaphoreType.DMA,
            pltpu.SemaphoreType.DMA,
            pltpu.SemaphoreType.DMA,
        ],
    )
    return fn(packed, pos0, pos1)


def _gemm_body(bw_ref, act_ref, x_ref, w_ref, b_ref, y_ref):
    del bw_ref
    blk = pl.program_id(0)

    @pl.when(act_ref[blk] > 0)
    def _():
        u = jax.lax.bitcast_convert_type(x_ref[...], jnp.uint32)
        x_lo = jax.lax.bitcast_convert_type(u << 16, jnp.float32)
        x_hi = jax.lax.bitcast_convert_type(
            u & jnp.uint32(0xFFFF0000), jnp.float32)
        w = w_ref[0]
        dn = (((1,), (1,)), ((), ()))
        acc = lax.dot_general(
            x_lo, w[:, :H], dn,
            preferred_element_type=jnp.float32,
            precision=lax.Precision.DEFAULT,
        ) + lax.dot_general(
            x_hi, w[:, H:], dn,
            preferred_element_type=jnp.float32,
            precision=lax.Precision.DEFAULT,
        ) + b_ref[0]
        lo = jax.lax.bitcast_convert_type(
            acc[:, :H].astype(jnp.bfloat16).astype(jnp.float32), jnp.uint32)
        hi = jax.lax.bitcast_convert_type(
            acc[:, H:].astype(jnp.bfloat16).astype(jnp.float32), jnp.uint32)
        y_ref[...] = ((lo >> 16) | hi).astype(jnp.int32)


def _grouped_gemm(Xg, We, be, blk_weight, blk_active):
    grid_spec = pltpu.PrefetchScalarGridSpec(
        num_scalar_prefetch=2,
        grid=(NB,),
        in_specs=[
            pl.BlockSpec((TILE, H), lambda b, s, a: (b, 0)),
            pl.BlockSpec((1, D, D), lambda b, s, a: (s[b], 0, 0)),
            pl.BlockSpec((1, 1, D), lambda b, s, a: (s[b], 0, 0)),
        ],
        out_specs=pl.BlockSpec((TILE, H), lambda b, s, a: (b, 0)),
    )
    return pl.pallas_call(
        _gemm_body,
        grid_spec=grid_spec,
        out_shape=jax.ShapeDtypeStruct((MPAD, H), jnp.int32),
    )(blk_weight, blk_active, Xg, We, be.reshape(E, 1, D))


def _combine_body(y_hbm, pos_hbm, g_hbm, out_hbm,
                  idx0, idx1, gv, rows_v, out_v, sem0, sem1):
    wid = lax.axis_index("s") * 2 + lax.axis_index("c")
    base = wid * TWORK
    idxs = (idx0, idx1)
    sems = (sem0, sem1)
    pltpu.sync_copy(g_hbm.at[pl.ds(K * base, K * TWORK)], gv)
    pend = [None, None]
    pltpu.sync_copy(pos_hbm.at[pl.ds(K * base, K * CCH)], idx0)
    pend[0] = pltpu.async_copy(y_hbm.at[idx0], rows_v.at[0], sem0)
    for c in range(CNCH):
        b = c % 2
        nb = (c + 1) % 2
        if c + 1 < CNCH:
            pltpu.sync_copy(
                pos_hbm.at[pl.ds(K * (base + (c + 1) * CCH), K * CCH)],
                idxs[nb])
            pend[nb] = pltpu.async_copy(
                y_hbm.at[idxs[nb]], rows_v.at[nb], sems[nb])
        pend[b].wait()
        gvec = gv[pl.ds(K * c * CCH, K * CCH)]

        def jbody(j, carry):
            off = j * 16
            for t in range(CCH):
                g0 = gvec[2 * t]
                g1 = gvec[2 * t + 1]
                ai = rows_v[b, 2 * t, pl.ds(off, 16)]
                bi = rows_v[b, 2 * t + 1, pl.ds(off, 16)]
                a_lo = jax.lax.bitcast_convert_type(ai << 16, jnp.float32)
                b_lo = jax.lax.bitcast_convert_type(bi << 16, jnp.float32)
                a_hi = jax.lax.bitcast_convert_type(ai & jnp.int32(-65536), jnp.float32)
                b_hi = jax.lax.bitcast_convert_type(bi & jnp.int32(-65536), jnp.float32)
                out_v[b, t, pl.ds(off, 16)] = a_lo * g0 + b_lo * g1
                out_v[b, t, pl.ds(H + off, 16)] = a_hi * g0 + b_hi * g1
            return carry

        lax.fori_loop(0, H // 16, jbody, 0)
        pltpu.sync_copy(out_v.at[b], out_hbm.at[pl.ds(base + c * CCH, CCH)])


def _combine(Y, pos, gates):
    mesh = plsc.VectorSubcoreMesh(core_axis_name="c", subcore_axis_name="s")
    fn = pl.kernel(
        _combine_body,
        out_type=jax.ShapeDtypeStruct((N, D), jnp.float32),
        mesh=mesh,
        scratch_types=[
            pltpu.VMEM((K * CCH,), jnp.int32),
            pltpu.VMEM((K * CCH,), jnp.int32),
            pltpu.VMEM((K * TWORK,), jnp.float32),
            pltpu.VMEM((2, K * CCH, H), jnp.int32),
            pltpu.VMEM((2, CCH, D), jnp.float32),
            pltpu.SemaphoreType.DMA,
            pltpu.SemaphoreType.DMA,
        ],
    )
    return fn(Y, pos, gates)


def _route(idx2):
    e_flat = idx2.reshape(M)
    onehot = (e_flat[:, None] == jnp.arange(E, dtype=jnp.int32)[None, :])
    oh = onehot.astype(jnp.int32)
    counts = jnp.sum(oh, axis=0)
    rank = jnp.sum(jnp.where(onehot, jnp.cumsum(oh, axis=0) - oh, 0), axis=1)
    nblk = (counts + TILE - 1) // TILE
    cum = jnp.cumsum(nblk)
    blk_off = cum - nblk
    pos = blk_off[e_flat] * TILE + rank
    bids = jnp.arange(NB, dtype=jnp.int32)
    blk_exp = jnp.minimum(
        jnp.sum((bids[:, None] >= cum[None, :]).astype(jnp.int32), axis=1),
        E - 1)
    blk_active = (bids < cum[E - 1]).astype(jnp.int32)
    last_exp = jnp.max(jnp.where(counts > 0,
                                 jnp.arange(E, dtype=jnp.int32), 0))
    blk_weight = jnp.where(blk_active > 0, blk_exp, last_exp)
    pos2 = pos.reshape(N, K)
    return pos, pos2[:, 0], pos2[:, 1], blk_weight, blk_active


def kernel(x, y, We, be, Wg, bg):
    idx2, gates2, packed = _gating(x, y, Wg, bg)
    pos, pos0, pos1, blk_weight, blk_active = _route(idx2)
    Xg = _dispatch(packed, pos0, pos1)
    Y = _grouped_gemm(Xg, We, be, blk_weight, blk_active)
    return _combine(Y, pos, gates2.reshape(M))
